# baseline (speedup 1.0000x reference)
import functools

import jax
import jax.numpy as jnp
from jax import lax
from jax.experimental import pallas as pl
from jax.experimental.pallas import tpu as pltpu
from jax.experimental.pallas import tpu_sc as plsc

_B, _N, _V = 64, 8, 100000
_NW = 32
_L = 16
_W = 128

_GDN = lax.GatherDimensionNumbers(
    offset_dims=(), collapsed_slice_dims=(0,), start_index_map=(0,))


def _take16(v, idx):
    return lax.gather(v, idx[:, None], _GDN, (1,),
                      mode=lax.GatherScatterMode.PROMISE_IN_BOUNDS)


def _sc_accept_body(draft_hbm, verify_t_hbm, tok_hbm, u_hbm, acc_out, emit_out,
                    tok_v, u_v, qbuf, pbuf, acc_v, emit_v, semq, semp):
    wid = lax.axis_index("s") * 2 + lax.axis_index("c")
    base = wid * _L
    pltpu.sync_copy(tok_hbm.at[pl.ds(base, _L)], tok_v)
    pltpu.sync_copy(u_hbm.at[pl.ds(base, _L)], u_v)
    lane = lax.iota(jnp.int32, _L)
    tok = tok_v[...]
    start_v = jnp.bitwise_and(tok, jnp.int32(~(_W - 1)))
    bbase = pl.multiple_of(jnp.bitwise_and(2 * wid, jnp.int32(~7)), 8)
    copies = []
    for k in range(_L):
        onehot = jnp.where(lane == k, start_v, 0)
        start_k = pl.multiple_of(lax.reduce_max(onehot, axes=(0,)), _W)
        b_k = 2 * wid + (k // _N)
        i_k = k % _N
        copies.append(pltpu.async_copy(
            draft_hbm.at[b_k, pl.ds(0, _N), pl.ds(start_k, _W)],
            qbuf.at[k], semq))
        copies.append(pltpu.async_copy(
            verify_t_hbm.at[i_k, pl.ds(bbase, 8), pl.ds(start_k, _W)],
            pbuf.at[k], semp))
    for c in copies:
        c.wait()
    off = jnp.bitwise_and(tok, jnp.int32(_W - 1))
    row_q = lax.rem(lane, jnp.int32(_N))
    row_p = lax.rem(2 * wid, jnp.int32(8)) + lax.div(lane, jnp.int32(_N))
    q = plsc.load_gather(qbuf, [lane, row_q, off])
    p = plsc.load_gather(pbuf, [lane, row_p, off])
    accept = u_v[...] < p / jnp.maximum(q, 1e-10)
    acc_i = jnp.where(accept, 1, 0)
    val = jnp.where(accept, 99, lax.rem(lane, jnp.int32(_N)))
    asum, vmin = acc_i, val
    for sh in (1, 2, 4):
        perm = lane ^ sh
        asum = asum + _take16(asum, perm)
        vmin = jnp.minimum(vmin, _take16(vmin, perm))
    emit = jnp.minimum(vmin, _N)
    sel = lax.rem(lane, jnp.int32(2)) * _N
    acc_v[...] = _take16(asum, sel)
    emit_v[...] = _take16(emit, sel)
    pltpu.sync_copy(acc_v, acc_out.at[wid])
    pltpu.sync_copy(emit_v, emit_out.at[wid])


_sc_accept = functools.partial(
    pl.kernel,
    mesh=plsc.VectorSubcoreMesh(core_axis_name="c", subcore_axis_name="s"),
    compiler_params=pltpu.CompilerParams(needs_layout_passes=False),
    out_type=(jax.ShapeDtypeStruct((_NW, _L), jnp.int32),
              jax.ShapeDtypeStruct((_NW, _L), jnp.int32)),
    scratch_types=[
        pltpu.VMEM((_L,), jnp.int32),
        pltpu.VMEM((_L,), jnp.float32),
        pltpu.VMEM((_L, _N, _W), jnp.float32),
        pltpu.VMEM((_L, _N, _W), jnp.float32),
        pltpu.VMEM((_L,), jnp.int32),
        pltpu.VMEM((_L,), jnp.int32),
        pltpu.SemaphoreType.DMA,
        pltpu.SemaphoreType.DMA,
    ],
)(_sc_accept_body)


_NBUF = 4


def _tc_sample_body(em_ref, jq_ref, verify_t_any, draft_any, eg_ref, ids_ref,
                    out_ref, pbuf, qbuf, psem, qsem):
    b = pl.program_id(0)

    def issue(bi):
        sl = lax.rem(bi, _NBUF)
        pltpu.make_async_copy(verify_t_any.at[em_ref[bi], pl.ds(bi, 1), :],
                              pbuf.at[pl.ds(sl, 1)], psem).start()
        pltpu.make_async_copy(draft_any.at[bi, pl.ds(jq_ref[bi], 1), :],
                              qbuf.at[pl.ds(sl, 1)], qsem).start()

    @pl.when(b == 0)
    def _():
        for k in range(_NBUF - 1):
            issue(k)

    @pl.when(b < _B - _NBUF + 1)
    def _():
        issue(b + _NBUF - 1)

    slot = lax.rem(b, _NBUF)
    pltpu.make_async_copy(verify_t_any.at[em_ref[b], pl.ds(b, 1), :],
                          pbuf.at[pl.ds(slot, 1)], psem).wait()
    pltpu.make_async_copy(draft_any.at[b, pl.ds(jq_ref[b], 1), :],
                          qbuf.at[pl.ds(slot, 1)], qsem).wait()

    em = em_ref[b]
    anyr = em < _N
    p = pbuf[pl.ds(slot, 1), :]
    q = qbuf[pl.ds(slot, 1), :]
    eg = eg_ref[pl.ds(lax.rem(b, 8), 1), :]
    resid = jnp.maximum(p - q, 0.0)
    s = jnp.sum(resid)
    final = jnp.where(anyr, resid / jnp.maximum(s, 1e-10), p)
    vals = (final + 1e-10) * eg
    m = jnp.max(vals)
    col = lax.broadcasted_iota(jnp.int32, (1, _V), 1)
    tok = jnp.min(jnp.where(vals == m, col, _V))
    dpad = ids_ref[0]
    pos = lax.broadcasted_iota(jnp.int32, (1, _N + 1), 1)
    res = jnp.where(pos < em, dpad, -1)
    res = jnp.where(pos == em, tok, res)
    out_ref[0] = res


def _tc_sample(em, jq, verify_t, draft_probs, eg, ids_pad):
    grid_spec = pltpu.PrefetchScalarGridSpec(
        num_scalar_prefetch=2,
        grid=(_B,),
        in_specs=[
            pl.BlockSpec(memory_space=pltpu.MemorySpace.HBM),
            pl.BlockSpec(memory_space=pltpu.MemorySpace.HBM),
            pl.BlockSpec((8, _V), lambda b, em, jq: (b // 8, 0)),
            pl.BlockSpec((1, 1, _N + 1), lambda b, em, jq: (b, 0, 0)),
        ],
        out_specs=pl.BlockSpec((1, 1, _N + 1), lambda b, em, jq: (b, 0, 0)),
        scratch_shapes=[
            pltpu.VMEM((_NBUF, _V), jnp.float32),
            pltpu.VMEM((_NBUF, _V), jnp.float32),
            pltpu.SemaphoreType.DMA,
            pltpu.SemaphoreType.DMA,
        ],
    )
    return pl.pallas_call(
        _tc_sample_body,
        grid_spec=grid_spec,
        out_shape=jax.ShapeDtypeStruct((_B, 1, _N + 1), jnp.int32),
    )(em, jq, verify_t, draft_probs, eg, ids_pad)


def kernel(draft_token_ids, draft_probs, verify_probs):
    rkey = jax.random.key(42)
    ku, ks = jax.random.split(rkey)
    u = jax.random.uniform(ku, (_B, _N), dtype=jnp.float32)
    tiny = float(jnp.finfo(jnp.float32).tiny)
    uu = jax.random.uniform(ks, (_B, _V), dtype=jnp.float32,
                            minval=tiny, maxval=1.0)
    eg = -1.0 / jnp.log(uu)

    verify_t = jnp.swapaxes(verify_probs, 0, 1)

    acc2d, emit2d = _sc_accept(draft_probs, verify_t,
                               draft_token_ids.reshape(-1), u.reshape(-1))
    accepted_num = acc2d[:, :2].reshape(_B)
    emitted_num = emit2d[:, :2].reshape(_B)
    jq = jnp.where(emitted_num < _N, emitted_num, 0)

    ids_pad = jnp.pad(draft_token_ids, ((0, 0), (0, 1)),
                      constant_values=-1).reshape(_B, 1, _N + 1)
    out3d = _tc_sample(emitted_num, jq, verify_t, draft_probs, eg, ids_pad)
    return out3d.reshape(_B, _N + 1), accepted_num, emitted_num


# speedup vs baseline: 1.0634x; 1.0634x over previous
import functools

import jax
import jax.numpy as jnp
from jax import lax
from jax.experimental import pallas as pl
from jax.experimental.pallas import tpu as pltpu
from jax.experimental.pallas import tpu_sc as plsc

_B, _N, _V = 64, 8, 100000
_NW = 32
_L = 16
_W = 128

_GDN = lax.GatherDimensionNumbers(
    offset_dims=(), collapsed_slice_dims=(0,), start_index_map=(0,))


def _take16(v, idx):
    return lax.gather(v, idx[:, None], _GDN, (1,),
                      mode=lax.GatherScatterMode.PROMISE_IN_BOUNDS)


def _sc_accept_body(draft_hbm, verify_t_hbm, tok_hbm, u_hbm, acc_out, emit_out,
                    tok_v, u_v, qbuf, pbuf, acc_v, emit_v, semq, semp):
    wid = lax.axis_index("s") * 2 + lax.axis_index("c")
    base = wid * _L
    pltpu.sync_copy(tok_hbm.at[pl.ds(base, _L)], tok_v)
    pltpu.sync_copy(u_hbm.at[pl.ds(base, _L)], u_v)
    lane = lax.iota(jnp.int32, _L)
    tok = tok_v[...]
    start_v = jnp.bitwise_and(tok, jnp.int32(~(_W - 1)))
    bbase = pl.multiple_of(jnp.bitwise_and(2 * wid, jnp.int32(~7)), 8)
    copies = []
    for k in range(_L):
        onehot = jnp.where(lane == k, start_v, 0)
        start_k = pl.multiple_of(lax.reduce_max(onehot, axes=(0,)), _W)
        b_k = 2 * wid + (k // _N)
        i_k = k % _N
        copies.append(pltpu.async_copy(
            draft_hbm.at[b_k, pl.ds(0, _N), pl.ds(start_k, _W)],
            qbuf.at[k], semq))
        copies.append(pltpu.async_copy(
            verify_t_hbm.at[i_k, pl.ds(bbase, 8), pl.ds(start_k, _W)],
            pbuf.at[k], semp))
    for c in copies:
        c.wait()
    off = jnp.bitwise_and(tok, jnp.int32(_W - 1))
    row_q = lax.rem(lane, jnp.int32(_N))
    row_p = lax.rem(2 * wid, jnp.int32(8)) + lax.div(lane, jnp.int32(_N))
    q = plsc.load_gather(qbuf, [lane, row_q, off])
    p = plsc.load_gather(pbuf, [lane, row_p, off])
    accept = u_v[...] < p / jnp.maximum(q, 1e-10)
    acc_i = jnp.where(accept, 1, 0)
    val = jnp.where(accept, 99, lax.rem(lane, jnp.int32(_N)))
    asum, vmin = acc_i, val
    for sh in (1, 2, 4):
        perm = lane ^ sh
        asum = asum + _take16(asum, perm)
        vmin = jnp.minimum(vmin, _take16(vmin, perm))
    emit = jnp.minimum(vmin, _N)
    sel = lax.rem(lane, jnp.int32(2)) * _N
    acc_v[...] = _take16(asum, sel)
    emit_v[...] = _take16(emit, sel)
    pltpu.sync_copy(acc_v, acc_out.at[wid])
    pltpu.sync_copy(emit_v, emit_out.at[wid])


_sc_accept = functools.partial(
    pl.kernel,
    mesh=plsc.VectorSubcoreMesh(core_axis_name="c", subcore_axis_name="s"),
    compiler_params=pltpu.CompilerParams(needs_layout_passes=False),
    out_type=(jax.ShapeDtypeStruct((_NW, _L), jnp.int32),
              jax.ShapeDtypeStruct((_NW, _L), jnp.int32)),
    scratch_types=[
        pltpu.VMEM((_L,), jnp.int32),
        pltpu.VMEM((_L,), jnp.float32),
        pltpu.VMEM((_L, _N, _W), jnp.float32),
        pltpu.VMEM((_L, _N, _W), jnp.float32),
        pltpu.VMEM((_L,), jnp.int32),
        pltpu.VMEM((_L,), jnp.int32),
        pltpu.SemaphoreType.DMA,
        pltpu.SemaphoreType.DMA,
    ],
)(_sc_accept_body)


_NBUF = 4
_NC = 8
_VC = _V // _NC


def _tc_sample_body(em_ref, jq_ref, verify_t_any, draft_any, eg_ref, colf_ref,
                    ids_ref, out_ref, pbuf, qbuf, psem, qsem):
    b = pl.program_id(0)

    def copies(bi, sl):
        return [
            pltpu.make_async_copy(
                verify_t_any.at[em_ref[bi], pl.ds(bi, 1), :],
                pbuf.at[pl.ds(sl, 1)], psem),
            pltpu.make_async_copy(
                draft_any.at[bi, pl.ds(jq_ref[bi], 1), :],
                qbuf.at[pl.ds(sl, 1)], qsem),
        ]

    def issue(bi):
        for c in copies(bi, lax.rem(bi, _NBUF)):
            c.start()

    @pl.when(b == 0)
    def _():
        for k in range(_NBUF - 1):
            issue(k)

    @pl.when(b < _B - _NBUF + 1)
    def _():
        issue(b + _NBUF - 1)

    slot = lax.rem(b, _NBUF)
    for c in copies(b, slot):
        c.wait()

    em = em_ref[b]
    anyr = em < _N
    p = jnp.stack([pbuf[slot, k * _VC:(k + 1) * _VC] for k in range(_NC)])
    q = jnp.stack([qbuf[slot, k * _VC:(k + 1) * _VC] for k in range(_NC)])
    eg = eg_ref[0]
    resid = jnp.maximum(p - q, 0.0)
    s = jnp.sum(resid)
    inv = jnp.where(anyr, 1.0 / jnp.maximum(s, 1e-10), 1.0)
    base = jnp.where(anyr, resid, p)
    vals = (base * inv + 1e-10) * eg
    m = jnp.max(vals)
    tokf = jnp.min(jnp.where(vals == m, colf_ref[...], jnp.float32(_V)))
    tok = tokf.astype(jnp.int32)
    dpad = ids_ref[0]
    pos = lax.broadcasted_iota(jnp.int32, (1, _N + 1), 1)
    res = jnp.where(pos < em, dpad, -1)
    res = jnp.where(pos == em, tok, res)
    out_ref[0] = res


def _tc_sample(em, jq, verify_t, draft_probs, eg, colf, ids_pad):
    grid_spec = pltpu.PrefetchScalarGridSpec(
        num_scalar_prefetch=2,
        grid=(_B,),
        in_specs=[
            pl.BlockSpec(memory_space=pltpu.MemorySpace.HBM),
            pl.BlockSpec(memory_space=pltpu.MemorySpace.HBM),
            pl.BlockSpec((1, _NC, _VC), lambda b, em, jq: (b, 0, 0)),
            pl.BlockSpec((_NC, _VC), lambda b, em, jq: (0, 0)),
            pl.BlockSpec((1, 1, _N + 1), lambda b, em, jq: (b, 0, 0)),
        ],
        out_specs=pl.BlockSpec((1, 1, _N + 1), lambda b, em, jq: (b, 0, 0)),
        scratch_shapes=[
            pltpu.VMEM((_NBUF, _V), jnp.float32),
            pltpu.VMEM((_NBUF, _V), jnp.float32),
            pltpu.SemaphoreType.DMA,
            pltpu.SemaphoreType.DMA,
        ],
    )
    return pl.pallas_call(
        _tc_sample_body,
        grid_spec=grid_spec,
        out_shape=jax.ShapeDtypeStruct((_B, 1, _N + 1), jnp.int32),
    )(em, jq, verify_t, draft_probs, eg, colf, ids_pad)


def kernel(draft_token_ids, draft_probs, verify_probs):
    rkey = jax.random.key(42)
    ku, ks = jax.random.split(rkey)
    u = jax.random.uniform(ku, (_B, _N), dtype=jnp.float32)
    tiny = float(jnp.finfo(jnp.float32).tiny)
    uu = jax.random.uniform(ks, (_B, _NC, _VC), dtype=jnp.float32,
                            minval=tiny, maxval=1.0)
    eg = -1.0 / jnp.log(uu)

    verify_t = jnp.swapaxes(verify_probs, 0, 1)

    acc2d, emit2d = _sc_accept(draft_probs, verify_t,
                               draft_token_ids.reshape(-1), u.reshape(-1))
    accepted_num = acc2d[:, :2].reshape(_B)
    emitted_num = emit2d[:, :2].reshape(_B)
    jq = jnp.where(emitted_num < _N, emitted_num, 0)

    ids_pad = jnp.pad(draft_token_ids, ((0, 0), (0, 1)),
                      constant_values=-1).reshape(_B, 1, _N + 1)
    colf = jnp.arange(_V, dtype=jnp.float32).reshape(_NC, _VC)
    out3d = _tc_sample(emitted_num, jq, verify_t, draft_probs, eg, colf,
                       ids_pad)
    return out3d.reshape(_B, _N + 1), accepted_num, emitted_num


# speedup vs baseline: 1.0670x; 1.0034x over previous
import functools

import jax
import jax.numpy as jnp
from jax import lax
from jax.experimental import pallas as pl
from jax.experimental.pallas import tpu as pltpu
from jax.experimental.pallas import tpu_sc as plsc

_B, _N, _V = 64, 8, 100000
_NW = 32
_L = 16
_W = 128

_GDN = lax.GatherDimensionNumbers(
    offset_dims=(), collapsed_slice_dims=(0,), start_index_map=(0,))


def _take16(v, idx):
    return lax.gather(v, idx[:, None], _GDN, (1,),
                      mode=lax.GatherScatterMode.PROMISE_IN_BOUNDS)


def _sc_accept_body(draft_hbm, verify_t_hbm, tok_hbm, u_hbm, acc_out, emit_out,
                    tok_v, u_v, qbuf, pbuf, acc_v, emit_v, semq, semp):
    wid = lax.axis_index("s") * 2 + lax.axis_index("c")
    base = wid * _L
    pltpu.sync_copy(tok_hbm.at[pl.ds(base, _L)], tok_v)
    pltpu.sync_copy(u_hbm.at[pl.ds(base, _L)], u_v)
    lane = lax.iota(jnp.int32, _L)
    tok = tok_v[...]
    start_v = jnp.bitwise_and(tok, jnp.int32(~(_W - 1)))
    bbase = pl.multiple_of(jnp.bitwise_and(2 * wid, jnp.int32(~7)), 8)
    copies = []
    for k in range(_L):
        onehot = jnp.where(lane == k, start_v, 0)
        start_k = pl.multiple_of(lax.reduce_max(onehot, axes=(0,)), _W)
        b_k = 2 * wid + (k // _N)
        i_k = k % _N
        copies.append(pltpu.async_copy(
            draft_hbm.at[b_k, pl.ds(0, _N), pl.ds(start_k, _W)],
            qbuf.at[k], semq))
        copies.append(pltpu.async_copy(
            verify_t_hbm.at[i_k, pl.ds(bbase, 8), pl.ds(start_k, _W)],
            pbuf.at[k], semp))
    for c in copies:
        c.wait()
    off = jnp.bitwise_and(tok, jnp.int32(_W - 1))
    row_q = lax.rem(lane, jnp.int32(_N))
    row_p = lax.rem(2 * wid, jnp.int32(8)) + lax.div(lane, jnp.int32(_N))
    q = plsc.load_gather(qbuf, [lane, row_q, off])
    p = plsc.load_gather(pbuf, [lane, row_p, off])
    accept = u_v[...] < p / jnp.maximum(q, 1e-10)
    acc_i = jnp.where(accept, 1, 0)
    val = jnp.where(accept, 99, lax.rem(lane, jnp.int32(_N)))
    asum, vmin = acc_i, val
    for sh in (1, 2, 4):
        perm = lane ^ sh
        asum = asum + _take16(asum, perm)
        vmin = jnp.minimum(vmin, _take16(vmin, perm))
    emit = jnp.minimum(vmin, _N)
    sel = lax.rem(lane, jnp.int32(2)) * _N
    acc_v[...] = _take16(asum, sel)
    emit_v[...] = _take16(emit, sel)
    pltpu.sync_copy(acc_v, acc_out.at[wid])
    pltpu.sync_copy(emit_v, emit_out.at[wid])


def _sc_accept(*args):
    fn = functools.partial(
        pl.kernel,
        mesh=plsc.VectorSubcoreMesh(core_axis_name="c", subcore_axis_name="s"),
        compiler_params=pltpu.CompilerParams(needs_layout_passes=False),
        out_type=(jax.ShapeDtypeStruct((_NW, _L), jnp.int32),
                  jax.ShapeDtypeStruct((_NW, _L), jnp.int32)),
        scratch_types=[
            pltpu.VMEM((_L,), jnp.int32),
            pltpu.VMEM((_L,), jnp.float32),
            pltpu.VMEM((_L, _N, _W), jnp.float32),
            pltpu.VMEM((_L, _N, _W), jnp.float32),
            pltpu.VMEM((_L,), jnp.int32),
            pltpu.VMEM((_L,), jnp.int32),
            pltpu.SemaphoreType.DMA,
            pltpu.SemaphoreType.DMA,
        ],
    )(_sc_accept_body)
    return fn(*args)


_NBUF = 4
_NC = 8
_VC = _V // _NC


def _tc_sample_body(em_ref, jq_ref, verify_t_any, draft_any, eg_ref, colf_ref,
                    ids_ref, out_ref, pbuf, qbuf, psem, qsem):
    b = pl.program_id(0)

    def copies(bi, sl):
        return [
            pltpu.make_async_copy(
                verify_t_any.at[em_ref[bi], pl.ds(bi, 1), :],
                pbuf.at[pl.ds(sl, 1)], psem),
            pltpu.make_async_copy(
                draft_any.at[bi, pl.ds(jq_ref[bi], 1), :],
                qbuf.at[pl.ds(sl, 1)], qsem),
        ]

    def issue(bi):
        for c in copies(bi, lax.rem(bi, _NBUF)):
            c.start()

    @pl.when(b == 0)
    def _():
        for k in range(_NBUF - 1):
            issue(k)

    @pl.when(b < _B - _NBUF + 1)
    def _():
        issue(b + _NBUF - 1)

    slot = lax.rem(b, _NBUF)
    for c in copies(b, slot):
        c.wait()

    em = em_ref[b]
    anyr = em < _N
    p = jnp.stack([pbuf[slot, k * _VC:(k + 1) * _VC] for k in range(_NC)])
    q = jnp.stack([qbuf[slot, k * _VC:(k + 1) * _VC] for k in range(_NC)])
    eg = eg_ref[0]
    resid = jnp.maximum(p - q, 0.0)
    s = jnp.sum(resid)
    inv = jnp.where(anyr, 1.0 / jnp.maximum(s, 1e-10), 1.0)
    base = jnp.where(anyr, resid, p)
    vals = (base * inv + 1e-10) * eg
    m = jnp.max(vals)
    tokf = jnp.min(jnp.where(vals == m, colf_ref[...], jnp.float32(_V)))
    tok = tokf.astype(jnp.int32)
    dpad = ids_ref[0]
    pos = lax.broadcasted_iota(jnp.int32, (1, _N + 1), 1)
    res = jnp.where(pos < em, dpad, -1)
    res = jnp.where(pos == em, tok, res)
    out_ref[0] = res


def _tc_sample(em, jq, verify_t, draft_probs, eg, colf, ids_pad):
    grid_spec = pltpu.PrefetchScalarGridSpec(
        num_scalar_prefetch=2,
        grid=(_B,),
        in_specs=[
            pl.BlockSpec(memory_space=pltpu.MemorySpace.HBM),
            pl.BlockSpec(memory_space=pltpu.MemorySpace.HBM),
            pl.BlockSpec((1, _NC, _VC), lambda b, em, jq: (b, 0, 0)),
            pl.BlockSpec((_NC, _VC), lambda b, em, jq: (0, 0)),
            pl.BlockSpec((1, 1, _N + 1), lambda b, em, jq: (b, 0, 0)),
        ],
        out_specs=pl.BlockSpec((1, 1, _N + 1), lambda b, em, jq: (b, 0, 0)),
        scratch_shapes=[
            pltpu.VMEM((_NBUF, _V), jnp.float32),
            pltpu.VMEM((_NBUF, _V), jnp.float32),
            pltpu.SemaphoreType.DMA,
            pltpu.SemaphoreType.DMA,
        ],
    )
    return pl.pallas_call(
        _tc_sample_body,
        grid_spec=grid_spec,
        out_shape=jax.ShapeDtypeStruct((_B, 1, _N + 1), jnp.int32),
    )(em, jq, verify_t, draft_probs, eg, colf, ids_pad)


def kernel(draft_token_ids, draft_probs, verify_probs):
    rkey = jax.random.key(42)
    ku, ks = jax.random.split(rkey)
    u = jax.random.uniform(ku, (_B, _N), dtype=jnp.float32)
    tiny = float(jnp.finfo(jnp.float32).tiny)
    uu = jax.random.uniform(ks, (_B, _NC, _VC), dtype=jnp.float32,
                            minval=tiny, maxval=1.0)
    eg = -1.0 / jnp.log(uu)

    verify_t = jnp.swapaxes(verify_probs, 0, 1)

    acc2d, emit2d = _sc_accept(draft_probs, verify_t,
                               draft_token_ids.reshape(-1), u.reshape(-1))
    accepted_num = acc2d[:, :2].reshape(_B)
    emitted_num = emit2d[:, :2].reshape(_B)
    jq = jnp.where(emitted_num < _N, emitted_num, 0)

    ids_pad = jnp.pad(draft_token_ids, ((0, 0), (0, 1)),
                      constant_values=-1).reshape(_B, 1, _N + 1)
    colf = jnp.arange(_V, dtype=jnp.float32).reshape(_NC, _VC)
    out3d = _tc_sample(emitted_num, jq, verify_t, draft_probs, eg, colf,
                       ids_pad)
    return out3d.reshape(_B, _N + 1), accepted_num, emitted_num
